# Initial kernel scaffold; baseline (speedup 1.0000x reference)
import jax
import jax.numpy as jnp
from jax.experimental import pallas as pl


def kernel(input_ids, table, W, b):
    raise NotImplementedError("write your pallas kernel here")


# speedup vs baseline: 1.1651x; 1.1651x over previous
import functools

import jax
import jax.numpy as jnp
from jax import lax
from jax.experimental import pallas as pl
from jax.experimental.pallas import tpu as pltpu
from jax.experimental.pallas import tpu_sc as plsc

_NC, _NS = 2, 16
_NW = _NC * _NS
_B, _L, _V, _D = 4096, 200, 16, 16
_N = _B * _L
_BPW = _N // _NW
_K = 8
_IW = 128
_CHUNK = _K * _IW
_STEPS = _BPW // _CHUNK


def _fused_table_body(t_ref, wt_ref, b_ref, o_ref):
    o_ref[...] = (
        jnp.dot(t_ref[...], wt_ref[...], preferred_element_type=jnp.float32)
        + b_ref[...]
    )


def _make_fused_table(table, WT, b2):
    return pl.pallas_call(
        _fused_table_body,
        out_shape=jax.ShapeDtypeStruct((_V, _D), jnp.float32),
    )(table, WT, b2)


_sc_mesh = plsc.VectorSubcoreMesh(core_axis_name="c", subcore_axis_name="s")


@functools.partial(
    pl.kernel,
    out_type=jax.ShapeDtypeStruct((_N, _D), jnp.float32),
    mesh=_sc_mesh,
    scratch_types=[
        pltpu.VMEM((_K, _IW), jnp.int32),
        pltpu.VMEM((_CHUNK, _D), jnp.float32),
        pltpu.SemaphoreType.DMA,
    ],
    compiler_params=pltpu.CompilerParams(use_tc_tiling_on_sc=False),
)
def _sc_gather(fused_hbm, ids_hbm, out_hbm, idx_v, rows_v, sem):
    wid = lax.axis_index("s") * _NC + lax.axis_index("c")
    row0 = wid * (_BPW // _IW)

    @pl.loop(0, _STEPS)
    def _chunk(ci):
        pltpu.sync_copy(ids_hbm.at[pl.ds(row0 + ci * _K, _K)], idx_v)
        copies = [
            pltpu.async_copy(
                fused_hbm.at[idx_v.at[j]],
                rows_v.at[pl.ds(j * _IW, _IW)],
                sem,
            )
            for j in range(_K)
        ]
        for c in copies:
            c.wait()
        pltpu.sync_copy(
            rows_v, out_hbm.at[pl.ds(wid * _BPW + ci * _CHUNK, _CHUNK)]
        )


def kernel(input_ids, table, W, b):
    ids = input_ids.reshape(_N // _IW, _IW).astype(jnp.int32)
    fused = _make_fused_table(table, W.T, b.reshape(1, _D))
    out = _sc_gather(fused, ids)
    return out.reshape(_B, _L, _D)


# speedup vs baseline: 4.8287x; 4.1444x over previous
import functools

import jax
import jax.numpy as jnp
from jax import lax
from jax.experimental import pallas as pl
from jax.experimental.pallas import tpu as pltpu
from jax.experimental.pallas import tpu_sc as plsc

_NC, _NS = 2, 16
_NW = _NC * _NS
_B, _L, _V, _D = 4096, 200, 16, 16
_N = _B * _L
_BPW = _N // _NW
_CHUNK = 2560
_NCHUNK = _BPW // _CHUNK
_NGRP = _CHUNK // 16


def _fused_table_body(t_ref, wt_ref, b_ref, o_ref):
    o_ref[...] = (
        jnp.dot(t_ref[...], wt_ref[...], preferred_element_type=jnp.float32)
        + b_ref[...]
    )


def _make_fused_table(table, WT, b2):
    return pl.pallas_call(
        _fused_table_body,
        out_shape=jax.ShapeDtypeStruct((_V, _D), jnp.float32),
    )(table, WT, b2)


_sc_mesh = plsc.VectorSubcoreMesh(core_axis_name="c", subcore_axis_name="s")


@functools.partial(
    pl.kernel,
    out_type=jax.ShapeDtypeStruct((_N * _D,), jnp.float32),
    mesh=_sc_mesh,
    scratch_types=[
        pltpu.VMEM((_V * _D,), jnp.float32),
        pltpu.VMEM((_BPW,), jnp.int32),
        pltpu.VMEM((_CHUNK * _D,), jnp.float32),
        pltpu.VMEM((_CHUNK * _D,), jnp.float32),
        pltpu.SemaphoreType.DMA,
        pltpu.SemaphoreType.DMA,
    ],
    compiler_params=pltpu.CompilerParams(
        use_tc_tiling_on_sc=False, needs_layout_passes=False
    ),
)
def _sc_gather(fused_hbm, ids_hbm, out_hbm, fused_v, ids_v, rows0_v,
               rows1_v, sem0, sem1):
    wid = lax.axis_index("s") * _NC + lax.axis_index("c")
    base = wid * _BPW
    pltpu.sync_copy(fused_hbm, fused_v)
    pltpu.sync_copy(ids_hbm.at[pl.ds(base, _BPW)], ids_v)

    iota = lax.iota(jnp.int32, 16)
    rot = [lax.rem(iota + c, 16) for c in range(16)]
    stc = [iota * 16 + r for r in rot]

    rows = (rows0_v, rows1_v)
    sems = (sem0, sem1)
    descs = [None, None]
    for ci in range(_NCHUNK):
        bu = ci % 2
        if descs[bu] is not None:
            descs[bu].wait()
        rows_v = rows[bu]

        @pl.loop(0, _NGRP)
        def _grp(g, ci=ci, rows_v=rows_v):
            ids16 = ids_v[pl.ds(ci * _CHUNK + g * 16, 16)]
            ldb = ids16 * 16
            stb = jnp.full((16,), g * 256, jnp.int32)
            for c in range(16):
                col = plsc.load_gather(fused_v, [ldb + rot[c]])
                plsc.store_scatter(rows_v, [stc[c] + stb], col)

        descs[bu] = pltpu.async_copy(
            rows_v,
            out_hbm.at[pl.ds((base + ci * _CHUNK) * _D, _CHUNK * _D)],
            sems[bu],
        )
    for d in descs:
        d.wait()


def kernel(input_ids, table, W, b):
    ids = input_ids.reshape(_N).astype(jnp.int32)
    fused = _make_fused_table(table, W.T, b.reshape(1, _D)).reshape(_V * _D)
    out = _sc_gather(fused, ids)
    return out.reshape(_B, _L, _D)


# speedup vs baseline: 5.6838x; 1.1771x over previous
import functools

import jax
import jax.numpy as jnp
from jax import lax
from jax.experimental import pallas as pl
from jax.experimental.pallas import tpu as pltpu
from jax.experimental.pallas import tpu_sc as plsc

_NC, _NS = 2, 16
_NW = _NC * _NS
_B, _L, _V, _D = 4096, 200, 16, 16
_N = _B * _L
_BPW = _N // _NW
_CHUNK = 2560
_NCHUNK = _BPW // _CHUNK
_NGRP = _CHUNK // 16


def _fused_table_body(t_ref, wt_ref, b_ref, o_ref):
    o_ref[...] = (
        jnp.dot(t_ref[...], wt_ref[...], preferred_element_type=jnp.float32)
        + b_ref[...]
    )


def _make_fused_table(table, WT, b2):
    return pl.pallas_call(
        _fused_table_body,
        out_shape=jax.ShapeDtypeStruct((_V, _D), jnp.float32),
    )(table, WT, b2)


_sc_mesh = plsc.VectorSubcoreMesh(core_axis_name="c", subcore_axis_name="s")


@functools.partial(
    pl.kernel,
    out_type=jax.ShapeDtypeStruct((_N * _D,), jnp.float32),
    mesh=_sc_mesh,
    scratch_types=[
        pltpu.VMEM((_V * _D,), jnp.float32),
        pltpu.VMEM((_BPW,), jnp.int32),
        pltpu.VMEM((_CHUNK * _D,), jnp.float32),
        pltpu.VMEM((_CHUNK * _D,), jnp.float32),
        pltpu.SemaphoreType.DMA,
        pltpu.SemaphoreType.DMA,
    ],
    compiler_params=pltpu.CompilerParams(
        use_tc_tiling_on_sc=False, needs_layout_passes=False
    ),
)
def _sc_gather(fused_hbm, ids_hbm, out_hbm, fused_v, ids_v, rows0_v,
               rows1_v, sem0, sem1):
    wid = lax.axis_index("s") * _NC + lax.axis_index("c")
    base = wid * _BPW
    pltpu.sync_copy(fused_hbm, fused_v)
    pltpu.sync_copy(ids_hbm.at[pl.ds(base, _BPW)], ids_v)

    iota = lax.iota(jnp.int32, 16)
    rot = [lax.rem(iota + c, 16) for c in range(16)]
    stc = [iota * 16 + r for r in rot]

    rows = (rows0_v, rows1_v)
    sems = (sem0, sem1)
    descs = [None, None]
    for ci in range(_NCHUNK):
        bu = ci % 2
        if descs[bu] is not None:
            descs[bu].wait()
        rows_v = rows[bu]

        @plsc.parallel_loop(0, _NGRP, unroll=2)
        def _grp(g, ci=ci, rows_v=rows_v):
            ids16 = ids_v[pl.ds(ci * _CHUNK + g * 16, 16)]
            ldb = ids16 * 16
            stb = jnp.full((16,), g * 256, jnp.int32)
            for c in range(16):
                col = plsc.load_gather(fused_v, [ldb + rot[c]])
                plsc.store_scatter(rows_v, [stc[c] + stb], col)

        descs[bu] = pltpu.async_copy(
            rows_v,
            out_hbm.at[pl.ds((base + ci * _CHUNK) * _D, _CHUNK * _D)],
            sems[bu],
        )
    for d in descs:
        d.wait()


def kernel(input_ids, table, W, b):
    ids = input_ids.reshape(_N).astype(jnp.int32)
    fused = _make_fused_table(table, W.T, b.reshape(1, _D)).reshape(_V * _D)
    out = _sc_gather(fused, ids)
    return out.reshape(_B, _L, _D)


# speedup vs baseline: 46.0233x; 8.0973x over previous
import functools

import jax
import jax.numpy as jnp
from jax import lax
from jax.experimental import pallas as pl
from jax.experimental.pallas import tpu as pltpu
from jax.experimental.pallas import tpu_sc as plsc

_NC, _NS = 2, 16
_NW = _NC * _NS
_B, _L, _V, _D = 4096, 200, 16, 16
_BW = _B // _NW
_LC = 20
_NCHUNK = _L // _LC
_NBG = _BW // 16


def _fused_table_body(w_ref, tt_ref, b_ref, o_ref):
    o_ref[...] = (
        jnp.dot(w_ref[...], tt_ref[...], preferred_element_type=jnp.float32)
        + b_ref[...]
    )


def _make_fused_table_t(W, tableT, b2):
    return pl.pallas_call(
        _fused_table_body,
        out_shape=jax.ShapeDtypeStruct((_D, _V), jnp.float32),
    )(W, tableT, b2)


_sc_mesh = plsc.VectorSubcoreMesh(core_axis_name="c", subcore_axis_name="s")


@functools.partial(
    pl.kernel,
    out_type=jax.ShapeDtypeStruct((_L * _D, _B), jnp.float32),
    mesh=_sc_mesh,
    scratch_types=[
        pltpu.VMEM((_V * _D,), jnp.float32),
        pltpu.VMEM((_L, _BW), jnp.int32),
        pltpu.VMEM((_LC * _D, _BW), jnp.float32),
        pltpu.VMEM((_LC * _D, _BW), jnp.float32),
        pltpu.SemaphoreType.DMA,
        pltpu.SemaphoreType.DMA,
    ],
    compiler_params=pltpu.CompilerParams(needs_layout_passes=False),
)
def _sc_gather(fused_hbm, ids_hbm, out_hbm, fused_v, ids_v, buf0_v, buf1_v,
               sem0, sem1):
    wid = lax.axis_index("s") * _NC + lax.axis_index("c")
    col0 = wid * _BW
    pltpu.sync_copy(fused_hbm, fused_v)
    pltpu.sync_copy(ids_hbm.at[:, pl.ds(col0, _BW)], ids_v)

    bufs = (buf0_v, buf1_v)
    sems = (sem0, sem1)

    @pl.loop(0, _NCHUNK // 2)
    def _pair(di):
        for half in range(2):
            ci = di * 2 + half
            buf_v = bufs[half]

            @pl.when(di > 0)
            def _drain(half=half, buf_v=buf_v):
                pltpu.make_async_copy(
                    out_hbm.at[pl.ds(0, _LC * _D), pl.ds(col0, _BW)],
                    buf_v,
                    sems[half],
                ).wait()

            @plsc.parallel_loop(0, _LC, unroll=1)
            def _pos(i, ci=ci, buf_v=buf_v):
                l = ci * _LC + i
                for bg in range(_NBG):
                    idsv = ids_v[l, pl.ds(bg * 16, 16)]
                    for d in range(_D):
                        col = plsc.load_gather(fused_v, [idsv + d * 16])
                        buf_v[i * _D + d, pl.ds(bg * 16, 16)] = col

            pltpu.async_copy(
                buf_v,
                out_hbm.at[
                    pl.ds(ci * (_LC * _D), _LC * _D), pl.ds(col0, _BW)
                ],
                sems[half],
            )

    for half in range(2):
        pltpu.make_async_copy(
            out_hbm.at[pl.ds(0, _LC * _D), pl.ds(col0, _BW)],
            bufs[half],
            sems[half],
        ).wait()


def kernel(input_ids, table, W, b):
    ids_t = input_ids.T.astype(jnp.int32)
    fused_t = _make_fused_table_t(W, table.T, b.reshape(_D, 1))
    out = _sc_gather(fused_t.reshape(_V * _D), ids_t)
    return out.reshape(_L, _D, _B).transpose(2, 0, 1)
